# baseline (speedup 1.0000x reference)
import functools

import jax
import jax.numpy as jnp
from jax.experimental import pallas as pl

BLK = 2048


def _fused_mlp_kernel(x_ref, cond_ref, w1_ref, wc_ref, b1_ref, w2_ref, b2_ref,
                      out_ref):
    h = jnp.dot(x_ref[...], w1_ref[...], preferred_element_type=jnp.float32)
    h += jnp.dot(cond_ref[...], wc_ref[...], preferred_element_type=jnp.float32)
    h += b1_ref[...]
    h = jnp.maximum(h, 0.0)
    out_ref[...] = (
        jnp.dot(h, w2_ref[...], preferred_element_type=jnp.float32)
        + b2_ref[...]
    )


@jax.jit
def kernel(x, cond, W1, Wc, b1, W2, b2):
    batch, input_dim = x.shape
    cond_dim = cond.shape[1]
    hidden = W1.shape[1]
    b1 = b1.reshape(1, hidden)
    b2 = b2.reshape(1, 1)
    grid = (batch // BLK,)
    return pl.pallas_call(
        _fused_mlp_kernel,
        grid=grid,
        in_specs=[
            pl.BlockSpec((BLK, input_dim), lambda i: (i, 0)),
            pl.BlockSpec((BLK, cond_dim), lambda i: (i, 0)),
            pl.BlockSpec((input_dim, hidden), lambda i: (0, 0)),
            pl.BlockSpec((cond_dim, hidden), lambda i: (0, 0)),
            pl.BlockSpec((1, hidden), lambda i: (0, 0)),
            pl.BlockSpec((hidden, 1), lambda i: (0, 0)),
            pl.BlockSpec((1, 1), lambda i: (0, 0)),
        ],
        out_specs=pl.BlockSpec((BLK, 1), lambda i: (i, 0)),
        out_shape=jax.ShapeDtypeStruct((batch, 1), jnp.float32),
    )(x, cond, W1, Wc, b1, W2, b2)
